# baseline (speedup 1.0000x reference)
import functools

import jax
import jax.numpy as jnp
from jax import lax
from jax.experimental import pallas as pl
from jax.experimental.pallas import tpu as pltpu
from jax.experimental.pallas import tpu_sc as plsc

N = 10000
E = 160000
D = 256
DH = D // 2

NUM_CORES = 2
NUM_SUBCORES = 16
NSEG = NUM_CORES * NUM_SUBCORES
HALF = N // NUM_CORES
SEG = E // NSEG
SEGCAP = SEG + 8
CHUNK = 80
ROWBLK = 8
NROWCHUNK = (HALF + ROWBLK - 1) // ROWBLK

def _ones16():
    return jnp.ones((16,), jnp.int32)


def _zeros16():
    return jnp.zeros((16,), jnp.int32)


def _proj_body(x_ref, wk_ref, wq_ref, wv_ref, ws_ref, bk_ref, bq_ref,
               bv_ref, bs_ref, bias_ref,
               k0_ref, k1_ref, q0_ref, q1_ref, v0_ref, v1_ref,
               s0_ref, s1_ref):
    xb = x_ref[...]
    k = jnp.dot(xb, wk_ref[...], preferred_element_type=jnp.float32) + bk_ref[...]
    q = jnp.dot(xb, wq_ref[...], preferred_element_type=jnp.float32) + bq_ref[...]
    v = jnp.dot(xb, wv_ref[...], preferred_element_type=jnp.float32) + bv_ref[...]
    s = (jnp.dot(xb, ws_ref[...], preferred_element_type=jnp.float32)
         + bs_ref[...] + bias_ref[...])
    k0_ref[...] = k[:, :DH]
    k1_ref[...] = k[:, DH:]
    q0_ref[...] = q[:, :DH]
    q1_ref[...] = q[:, DH:]
    v0_ref[...] = v[:, :DH]
    v1_ref[...] = v[:, DH:]
    s0_ref[...] = s[:, :DH]
    s1_ref[...] = s[:, DH:]


def _projections(x, wkT, wqT, wvT, wsT, bk, bq, bv, bs, bias):
    blk = 1000
    grid = (N // blk,)
    xspec = pl.BlockSpec((blk, D), lambda i: (i, 0))
    wspec = pl.BlockSpec((D, D), lambda i: (0, 0))
    bspec = pl.BlockSpec((1, D), lambda i: (0, 0))
    ospec = pl.BlockSpec((blk, DH), lambda i: (i, 0))
    oshape = jax.ShapeDtypeStruct((N, DH), jnp.float32)
    return pl.pallas_call(
        _proj_body,
        grid=grid,
        in_specs=[xspec, wspec, wspec, wspec, wspec,
                  bspec, bspec, bspec, bspec, bspec],
        out_specs=[ospec] * 8,
        out_shape=[oshape] * 8,
    )(x, wkT, wqT, wvT, wsT, bk, bq, bv, bs, bias)


def _part_body(src_hbm, dst_hbm, srcp_hbm, dstp_hbm, cnt_hbm,
               src_seg, dst_seg, osrc0, odst0, osrc1, odst1, cbuf):
    c = lax.axis_index("c")
    s = lax.axis_index("s")
    seg = c * NUM_SUBCORES + s
    e0 = seg * SEG

    pltpu.sync_copy(src_hbm.at[pl.ds(e0, SEGCAP)], src_seg)
    pltpu.sync_copy(dst_hbm.at[pl.ds(e0, SEGCAP)], dst_seg)

    def route(d, sv, valid, f0v, f1v):
        m0 = d < HALF
        m1 = d >= HALF
        if valid is not None:
            m0 = valid & m0
            m1 = valid & m1
        i0 = jnp.where(m0, _ones16(), _zeros16())
        i1 = jnp.where(m1, _ones16(), _zeros16())
        p0 = f0v + lax.cumsum(i0) - 1
        p1 = f1v + lax.cumsum(i1) - 1
        plsc.store_scatter(odst0, [p0], d, mask=m0)
        plsc.store_scatter(osrc0, [p0], sv, mask=m0)
        plsc.store_scatter(odst1, [p1], d - HALF, mask=m1)
        plsc.store_scatter(osrc1, [p1], sv, mask=m1)
        return (f0v + plsc.all_reduce_population_count(m0),
                f1v + plsc.all_reduce_population_count(m1))

    def step(i, carry):
        f0v, f1v = carry
        sl = pl.ds(i * 16, 16)
        return route(dst_seg[sl], src_seg[sl], None, f0v, f1v)

    nfull = SEG // 16
    f0v, f1v = lax.fori_loop(0, nfull, step, (_zeros16(), _zeros16()))

    tail = SEG - nfull * 16
    if tail:
        sl = pl.ds(nfull * 16, 16)
        valid = lax.iota(jnp.int32, 16) < tail
        f0v, f1v = route(dst_seg[sl], src_seg[sl], valid, f0v, f1v)

    pltpu.sync_copy(osrc0, srcp_hbm.at[pl.ds(seg * SEGCAP, SEGCAP)])
    pltpu.sync_copy(odst0, dstp_hbm.at[pl.ds(seg * SEGCAP, SEGCAP)])
    pltpu.sync_copy(osrc1, srcp_hbm.at[pl.ds((NSEG + seg) * SEGCAP, SEGCAP)])
    pltpu.sync_copy(odst1, dstp_hbm.at[pl.ds((NSEG + seg) * SEGCAP, SEGCAP)])
    cbuf[pl.ds(0, 16)] = f0v
    pltpu.sync_copy(cbuf, cnt_hbm.at[pl.ds(seg * 16, 16)])
    cbuf[pl.ds(0, 16)] = f1v
    pltpu.sync_copy(cbuf, cnt_hbm.at[pl.ds((NSEG + seg) * 16, 16)])


def _partition(src, dst):
    mesh = plsc.VectorSubcoreMesh(core_axis_name="c", subcore_axis_name="s",
                                  num_cores=NUM_CORES,
                                  num_subcores=NUM_SUBCORES)
    fn = pl.kernel(
        _part_body,
        compiler_params=pltpu.CompilerParams(needs_layout_passes=False),
        out_type=[
            jax.ShapeDtypeStruct((2 * NSEG * SEGCAP,), jnp.int32),
            jax.ShapeDtypeStruct((2 * NSEG * SEGCAP,), jnp.int32),
            jax.ShapeDtypeStruct((2 * NSEG * 16,), jnp.int32),
        ],
        mesh=mesh,
        scratch_types=[
            pltpu.VMEM((SEGCAP,), jnp.int32),
            pltpu.VMEM((SEGCAP,), jnp.int32),
            pltpu.VMEM((SEGCAP,), jnp.int32),
            pltpu.VMEM((SEGCAP,), jnp.int32),
            pltpu.VMEM((SEGCAP,), jnp.int32),
            pltpu.VMEM((SEGCAP,), jnp.int32),
            pltpu.VMEM((16,), jnp.int32),
        ],
    )
    return fn(src, dst)


def _edge_body(k0_hbm, q0_hbm, v0_hbm, s0_hbm, k1_hbm, q1_hbm, v1_hbm,
               s1_hbm, srcp_hbm, dstp_hbm, cnt_hbm,
               out0_hbm, out1_hbm, src_seg, dst_seg,
               idx_src0, idx_dstg0, idx_loc0, kbuf0, qbuf0, vbuf0,
               idx_src1, idx_dstg1, idx_loc1, kbuf1, qbuf1, vbuf1,
               mbuf, cbuf, acc, sem0, sem1):
    c = lax.axis_index("c")
    s = lax.axis_index("s")
    base = c * HALF

    lanes = [lax.iota(jnp.int32, 16) + j * 16 for j in range(CHUNK // 16)]
    sets = ((idx_src0, idx_dstg0, idx_loc0, kbuf0, qbuf0, vbuf0, sem0),
            (idx_src1, idx_dstg1, idx_loc1, kbuf1, qbuf1, vbuf1, sem1))

    def build(blk, remv, st):
        idx_src, idx_dstg, idx_loc = st[0], st[1], st[2]
        for j in range(CHUNK // 16):
            sl = pl.ds(blk * CHUNK + j * 16, 16)
            osl = pl.ds(j * 16, 16)
            valid = lanes[j] < remv
            sv = src_seg[sl]
            dv = dst_seg[sl]
            idx_src[osl] = jnp.where(valid, sv, 0)
            idx_dstg[osl] = jnp.where(valid, dv + base, 0)
            idx_loc[osl] = jnp.where(valid, dv, HALF)

    for half in range(2):
        k_hbm, q_hbm, v_hbm, skip_hbm, out_hbm = (
            (k0_hbm, q0_hbm, v0_hbm, s0_hbm, out0_hbm) if half == 0
            else (k1_hbm, q1_hbm, v1_hbm, s1_hbm, out1_hbm))

        def fire(st):
            idx_src, idx_dstg, st_sem = st[0], st[1], st[6]
            pltpu.make_async_copy(k_hbm.at[idx_dstg], st[3], st_sem).start()
            pltpu.make_async_copy(q_hbm.at[idx_src], st[4], st_sem).start()
            pltpu.make_async_copy(v_hbm.at[idx_src], st[5], st_sem).start()

        def wait3(st):
            idx_src, idx_dstg, st_sem = st[0], st[1], st[6]
            pltpu.make_async_copy(k_hbm.at[idx_dstg], st[3], st_sem).wait()
            pltpu.make_async_copy(q_hbm.at[idx_src], st[4], st_sem).wait()
            pltpu.make_async_copy(v_hbm.at[idx_src], st[5], st_sem).wait()

        def process(st):
            kbuf, qbuf, vbuf = st[3], st[4], st[5]

            @plsc.parallel_loop(0, CHUNK, unroll=4)
            def _(e):
                for j in range(DH // 16):
                    sl = pl.ds(j * 16, 16)
                    t = kbuf[e, sl] + qbuf[e, sl]
                    sig = 1.0 / (1.0 + jnp.exp(-t))
                    mbuf[e, sl] = sig * vbuf[e, sl]

            pltpu.sync_copy(mbuf, acc.at[st[2]], add=True)

        def init_step(t, _):
            chunk = s + t * NUM_SUBCORES

            @pl.when(chunk < NROWCHUNK)
            def _():
                pltpu.sync_copy(
                    skip_hbm.at[pl.ds(base + chunk * ROWBLK, ROWBLK)],
                    acc.at[pl.ds(chunk * ROWBLK, ROWBLK)])
            return 0

        lax.fori_loop(0, (NROWCHUNK + NUM_SUBCORES - 1) // NUM_SUBCORES,
                      init_step, 0)
        plsc.subcore_barrier()

        for t2 in range(2):
            seg = 2 * s + t2
            lbase = (c * NSEG + seg) * SEGCAP
            pltpu.sync_copy(srcp_hbm.at[pl.ds(lbase, SEGCAP)],
                            src_seg.at[pl.ds(0, SEGCAP)])
            pltpu.sync_copy(dstp_hbm.at[pl.ds(lbase, SEGCAP)],
                            dst_seg.at[pl.ds(0, SEGCAP)])
            pltpu.sync_copy(cnt_hbm.at[pl.ds((c * NSEG + seg) * 16, 16)], cbuf)
            cntv = cbuf[pl.ds(0, 16)]
            cnt = jnp.max(cntv)
            nblk = (cnt + (CHUNK - 1)) // CHUNK
            npair = (nblk + 1) // 2

            @pl.when(nblk > 0)
            def _():
                build(0, cntv, sets[0])
                fire(sets[0])

            def pair_step(p, remv):
                for h2 in range(2):
                    st = sets[h2]
                    other = sets[1 - h2]
                    blk = p * 2 + h2
                    rv = remv

                    @pl.when(blk + 1 < nblk)
                    def _():
                        build(blk + 1, rv, other)
                        fire(other)

                    @pl.when(blk < nblk)
                    def _():
                        wait3(st)
                        process(st)

                    remv = remv - CHUNK
                return remv

            lax.fori_loop(0, npair, pair_step, cntv - CHUNK)

        plsc.subcore_barrier()

        def out_step(t, _):
            chunk = s + t * NUM_SUBCORES

            @pl.when(chunk < NROWCHUNK)
            def _():
                pltpu.sync_copy(
                    acc.at[pl.ds(chunk * ROWBLK, ROWBLK)],
                    out_hbm.at[pl.ds(base + chunk * ROWBLK, ROWBLK)])
            return 0

        lax.fori_loop(0, (NROWCHUNK + NUM_SUBCORES - 1) // NUM_SUBCORES,
                      out_step, 0)
        if half == 0:
            plsc.subcore_barrier()


CAPBUF = ((SEG + CHUNK - 1) // CHUNK) * CHUNK


def _edge_aggregate(k0, q0, v0, s0, k1, q1, v1, s1, srcp, dstp, cnt):
    mesh = plsc.VectorSubcoreMesh(core_axis_name="c", subcore_axis_name="s",
                                  num_cores=NUM_CORES,
                                  num_subcores=NUM_SUBCORES)
    bufset = [
        pltpu.VMEM((CHUNK,), jnp.int32),
        pltpu.VMEM((CHUNK,), jnp.int32),
        pltpu.VMEM((CHUNK,), jnp.int32),
        pltpu.VMEM((CHUNK, DH), jnp.float32),
        pltpu.VMEM((CHUNK, DH), jnp.float32),
        pltpu.VMEM((CHUNK, DH), jnp.float32),
    ]
    fn = pl.kernel(
        _edge_body,
        compiler_params=pltpu.CompilerParams(needs_layout_passes=False),
        out_type=[jax.ShapeDtypeStruct((N, DH), jnp.float32),
                  jax.ShapeDtypeStruct((N, DH), jnp.float32)],
        mesh=mesh,
        scratch_types=(
            [pltpu.VMEM((CAPBUF,), jnp.int32),
             pltpu.VMEM((CAPBUF,), jnp.int32)]
            + bufset + bufset
            + [pltpu.VMEM((CHUNK, DH), jnp.float32),
               pltpu.VMEM((16,), jnp.int32),
               pltpu.VMEM_SHARED((HALF + ROWBLK, DH), jnp.float32),
               pltpu.SemaphoreType.DMA,
               pltpu.SemaphoreType.DMA]
        ),
    )
    return fn(k0, q0, v0, s0, k1, q1, v1, s1, srcp, dstp, cnt)


def kernel(x, edge_index, edge_attr, W_key, b_key, W_query, b_query,
           W_value, b_value, W_skip, b_skip, bias):
    del edge_attr
    k0, k1, q0, q1, v0, v1, s0, s1 = _projections(
        x, W_key.T, W_query.T, W_value.T, W_skip.T,
        b_key.reshape(1, D), b_query.reshape(1, D), b_value.reshape(1, D),
        b_skip.reshape(1, D), bias.reshape(1, D))
    src = jnp.pad(edge_index[0], (0, 16))
    dst = jnp.pad(edge_index[1], (0, 16))
    srcp, dstp, cnt = _partition(src, dst)
    out0, out1 = _edge_aggregate(k0, q0, v0, s0, k1, q1, v1, s1,
                                 srcp, dstp, cnt)
    return jnp.concatenate([out0, out1], axis=1)


# speedup vs baseline: 1.0545x; 1.0545x over previous
import functools

import jax
import jax.numpy as jnp
from jax import lax
from jax.experimental import pallas as pl
from jax.experimental.pallas import tpu as pltpu
from jax.experimental.pallas import tpu_sc as plsc

N = 10000
E = 160000
D = 256
DH = D // 2

NUM_CORES = 2
NUM_SUBCORES = 16
NSEG = NUM_CORES * NUM_SUBCORES
HALF = N // NUM_CORES
SEG = E // NSEG
SEGCAP = SEG + 8
CHUNK = 80
ROWBLK = 8
NROWCHUNK = (HALF + ROWBLK - 1) // ROWBLK

def _ones16():
    return jnp.ones((16,), jnp.int32)


def _zeros16():
    return jnp.zeros((16,), jnp.int32)


def _proj_body(x_ref, wk_ref, wq_ref, wv_ref, ws_ref, bk_ref, bq_ref,
               bv_ref, bs_ref, bias_ref,
               k0_ref, k1_ref, q0_ref, q1_ref, v0_ref, v1_ref,
               s0_ref, s1_ref):
    xb = x_ref[...]
    k = jnp.dot(xb, wk_ref[...], preferred_element_type=jnp.float32) + bk_ref[...]
    q = jnp.dot(xb, wq_ref[...], preferred_element_type=jnp.float32) + bq_ref[...]
    v = jnp.dot(xb, wv_ref[...], preferred_element_type=jnp.float32) + bv_ref[...]
    s = (jnp.dot(xb, ws_ref[...], preferred_element_type=jnp.float32)
         + bs_ref[...] + bias_ref[...])
    k0_ref[...] = k[:, :DH]
    k1_ref[...] = k[:, DH:]
    q0_ref[...] = q[:, :DH]
    q1_ref[...] = q[:, DH:]
    v0_ref[...] = v[:, :DH]
    v1_ref[...] = v[:, DH:]
    s0_ref[...] = s[:, :DH]
    s1_ref[...] = s[:, DH:]


def _projections(x, wkT, wqT, wvT, wsT, bk, bq, bv, bs, bias):
    blk = 1000
    grid = (N // blk,)
    xspec = pl.BlockSpec((blk, D), lambda i: (i, 0))
    wspec = pl.BlockSpec((D, D), lambda i: (0, 0))
    bspec = pl.BlockSpec((1, D), lambda i: (0, 0))
    ospec = pl.BlockSpec((blk, DH), lambda i: (i, 0))
    oshape = jax.ShapeDtypeStruct((N, DH), jnp.float32)
    return pl.pallas_call(
        _proj_body,
        grid=grid,
        in_specs=[xspec, wspec, wspec, wspec, wspec,
                  bspec, bspec, bspec, bspec, bspec],
        out_specs=[ospec] * 8,
        out_shape=[oshape] * 8,
    )(x, wkT, wqT, wvT, wsT, bk, bq, bv, bs, bias)


def _part_body(src_hbm, dst_hbm, srcp_hbm, dstp_hbm, cnt_hbm,
               src_seg, dst_seg, osrc0, odst0, osrc1, odst1, cbuf):
    c = lax.axis_index("c")
    s = lax.axis_index("s")
    seg = c * NUM_SUBCORES + s
    e0 = seg * SEG

    pltpu.sync_copy(src_hbm.at[pl.ds(e0, SEGCAP)], src_seg)
    pltpu.sync_copy(dst_hbm.at[pl.ds(e0, SEGCAP)], dst_seg)

    def route(d, sv, valid, f0v, f1v):
        m0 = d < HALF
        m1 = d >= HALF
        if valid is not None:
            m0 = valid & m0
            m1 = valid & m1
        i0 = jnp.where(m0, _ones16(), _zeros16())
        i1 = jnp.where(m1, _ones16(), _zeros16())
        p0 = f0v + lax.cumsum(i0) - 1
        p1 = f1v + lax.cumsum(i1) - 1
        plsc.store_scatter(odst0, [p0], d, mask=m0)
        plsc.store_scatter(osrc0, [p0], sv, mask=m0)
        plsc.store_scatter(odst1, [p1], d - HALF, mask=m1)
        plsc.store_scatter(osrc1, [p1], sv, mask=m1)
        return (f0v + plsc.all_reduce_population_count(m0),
                f1v + plsc.all_reduce_population_count(m1))

    def step(i, carry):
        f0v, f1v = carry
        sl = pl.ds(i * 16, 16)
        return route(dst_seg[sl], src_seg[sl], None, f0v, f1v)

    nfull = SEG // 16
    f0v, f1v = lax.fori_loop(0, nfull, step, (_zeros16(), _zeros16()))

    tail = SEG - nfull * 16
    if tail:
        sl = pl.ds(nfull * 16, 16)
        valid = lax.iota(jnp.int32, 16) < tail
        f0v, f1v = route(dst_seg[sl], src_seg[sl], valid, f0v, f1v)

    pltpu.sync_copy(osrc0, srcp_hbm.at[pl.ds(seg * SEGCAP, SEGCAP)])
    pltpu.sync_copy(odst0, dstp_hbm.at[pl.ds(seg * SEGCAP, SEGCAP)])
    pltpu.sync_copy(osrc1, srcp_hbm.at[pl.ds((NSEG + seg) * SEGCAP, SEGCAP)])
    pltpu.sync_copy(odst1, dstp_hbm.at[pl.ds((NSEG + seg) * SEGCAP, SEGCAP)])
    cbuf[pl.ds(0, 16)] = f0v
    pltpu.sync_copy(cbuf, cnt_hbm.at[pl.ds(seg * 16, 16)])
    cbuf[pl.ds(0, 16)] = f1v
    pltpu.sync_copy(cbuf, cnt_hbm.at[pl.ds((NSEG + seg) * 16, 16)])


def _partition(src, dst):
    mesh = plsc.VectorSubcoreMesh(core_axis_name="c", subcore_axis_name="s",
                                  num_cores=NUM_CORES,
                                  num_subcores=NUM_SUBCORES)
    fn = pl.kernel(
        _part_body,
        compiler_params=pltpu.CompilerParams(needs_layout_passes=False),
        out_type=[
            jax.ShapeDtypeStruct((2 * NSEG * SEGCAP,), jnp.int32),
            jax.ShapeDtypeStruct((2 * NSEG * SEGCAP,), jnp.int32),
            jax.ShapeDtypeStruct((2 * NSEG * 16,), jnp.int32),
        ],
        mesh=mesh,
        scratch_types=[
            pltpu.VMEM((SEGCAP,), jnp.int32),
            pltpu.VMEM((SEGCAP,), jnp.int32),
            pltpu.VMEM((SEGCAP,), jnp.int32),
            pltpu.VMEM((SEGCAP,), jnp.int32),
            pltpu.VMEM((SEGCAP,), jnp.int32),
            pltpu.VMEM((SEGCAP,), jnp.int32),
            pltpu.VMEM((16,), jnp.int32),
        ],
    )
    return fn(src, dst)


def _edge_body(k0_hbm, q0_hbm, v0_hbm, s0_hbm, k1_hbm, q1_hbm, v1_hbm,
               s1_hbm, srcp_hbm, dstp_hbm, cnt_hbm,
               out0_hbm, out1_hbm, src_seg, dst_seg,
               idx_src0, idx_dstg0, idx_loc0, kbuf0, qbuf0, vbuf0,
               idx_src1, idx_dstg1, idx_loc1, kbuf1, qbuf1, vbuf1,
               mbuf, cbuf, acc, sem0, sem1):
    c = lax.axis_index("c")
    s = lax.axis_index("s")
    base = c * HALF

    lanes = [lax.iota(jnp.int32, 16) + j * 16 for j in range(CHUNK // 16)]
    sets = ((idx_src0, idx_dstg0, idx_loc0, kbuf0, qbuf0, vbuf0, sem0),
            (idx_src1, idx_dstg1, idx_loc1, kbuf1, qbuf1, vbuf1, sem1))

    def build(blk, remv, st):
        idx_src, idx_dstg, idx_loc = st[0], st[1], st[2]
        for j in range(CHUNK // 16):
            sl = pl.ds(blk * CHUNK + j * 16, 16)
            osl = pl.ds(j * 16, 16)
            valid = lanes[j] < remv
            sv = src_seg[sl]
            dv = dst_seg[sl]
            idx_src[osl] = jnp.where(valid, sv, 0)
            idx_dstg[osl] = jnp.where(valid, dv + base, 0)
            idx_loc[osl] = jnp.where(valid, dv, HALF)

    for half in range(2):
        k_hbm, q_hbm, v_hbm, skip_hbm, out_hbm = (
            (k0_hbm, q0_hbm, v0_hbm, s0_hbm, out0_hbm) if half == 0
            else (k1_hbm, q1_hbm, v1_hbm, s1_hbm, out1_hbm))

        def fire(st):
            idx_src, idx_dstg, st_sem = st[0], st[1], st[6]
            pltpu.make_async_copy(k_hbm.at[idx_dstg], st[3], st_sem).start()
            pltpu.make_async_copy(q_hbm.at[idx_src], st[4], st_sem).start()
            pltpu.make_async_copy(v_hbm.at[idx_src], st[5], st_sem).start()

        def wait3(st):
            idx_src, idx_dstg, st_sem = st[0], st[1], st[6]
            pltpu.make_async_copy(k_hbm.at[idx_dstg], st[3], st_sem).wait()
            pltpu.make_async_copy(q_hbm.at[idx_src], st[4], st_sem).wait()
            pltpu.make_async_copy(v_hbm.at[idx_src], st[5], st_sem).wait()

        def process(st):
            kbuf, qbuf, vbuf = st[3], st[4], st[5]

            def gate_step(e, _):
                for j in range(DH // 16):
                    sl = pl.ds(j * 16, 16)
                    t = kbuf[e, sl] + qbuf[e, sl]
                    sig = 1.0 / (1.0 + jnp.exp(-t))
                    mbuf[e, sl] = sig * vbuf[e, sl]
                return 0

            lax.fori_loop(0, CHUNK, gate_step, 0)
            pltpu.sync_copy(mbuf, acc.at[st[2]], add=True)

        def init_step(t, _):
            chunk = s + t * NUM_SUBCORES

            @pl.when(chunk < NROWCHUNK)
            def _():
                pltpu.sync_copy(
                    skip_hbm.at[pl.ds(base + chunk * ROWBLK, ROWBLK)],
                    acc.at[pl.ds(chunk * ROWBLK, ROWBLK)])
            return 0

        lax.fori_loop(0, (NROWCHUNK + NUM_SUBCORES - 1) // NUM_SUBCORES,
                      init_step, 0)
        plsc.subcore_barrier()

        for t2 in range(2):
            seg = 2 * s + t2
            lbase = (c * NSEG + seg) * SEGCAP
            pltpu.sync_copy(srcp_hbm.at[pl.ds(lbase, SEGCAP)],
                            src_seg.at[pl.ds(0, SEGCAP)])
            pltpu.sync_copy(dstp_hbm.at[pl.ds(lbase, SEGCAP)],
                            dst_seg.at[pl.ds(0, SEGCAP)])
            pltpu.sync_copy(cnt_hbm.at[pl.ds((c * NSEG + seg) * 16, 16)], cbuf)
            cntv = cbuf[pl.ds(0, 16)]
            cnt = jnp.max(cntv)
            nblk = (cnt + (CHUNK - 1)) // CHUNK
            npair = (nblk + 1) // 2

            @pl.when(nblk > 0)
            def _():
                build(0, cntv, sets[0])
                fire(sets[0])

            def pair_step(p, remv):
                for h2 in range(2):
                    st = sets[h2]
                    other = sets[1 - h2]
                    blk = p * 2 + h2
                    rv = remv

                    @pl.when(blk + 1 < nblk)
                    def _():
                        build(blk + 1, rv, other)
                        fire(other)

                    @pl.when(blk < nblk)
                    def _():
                        wait3(st)
                        process(st)

                    remv = remv - CHUNK
                return remv

            lax.fori_loop(0, npair, pair_step, cntv - CHUNK)

        plsc.subcore_barrier()

        def out_step(t, _):
            chunk = s + t * NUM_SUBCORES

            @pl.when(chunk < NROWCHUNK)
            def _():
                pltpu.sync_copy(
                    acc.at[pl.ds(chunk * ROWBLK, ROWBLK)],
                    out_hbm.at[pl.ds(base + chunk * ROWBLK, ROWBLK)])
            return 0

        lax.fori_loop(0, (NROWCHUNK + NUM_SUBCORES - 1) // NUM_SUBCORES,
                      out_step, 0)
        if half == 0:
            plsc.subcore_barrier()


CAPBUF = ((SEG + CHUNK - 1) // CHUNK) * CHUNK


def _edge_aggregate(k0, q0, v0, s0, k1, q1, v1, s1, srcp, dstp, cnt):
    mesh = plsc.VectorSubcoreMesh(core_axis_name="c", subcore_axis_name="s",
                                  num_cores=NUM_CORES,
                                  num_subcores=NUM_SUBCORES)
    bufset = [
        pltpu.VMEM((CHUNK,), jnp.int32),
        pltpu.VMEM((CHUNK,), jnp.int32),
        pltpu.VMEM((CHUNK,), jnp.int32),
        pltpu.VMEM((CHUNK, DH), jnp.float32),
        pltpu.VMEM((CHUNK, DH), jnp.float32),
        pltpu.VMEM((CHUNK, DH), jnp.float32),
    ]
    fn = pl.kernel(
        _edge_body,
        compiler_params=pltpu.CompilerParams(needs_layout_passes=False),
        out_type=[jax.ShapeDtypeStruct((N, DH), jnp.float32),
                  jax.ShapeDtypeStruct((N, DH), jnp.float32)],
        mesh=mesh,
        scratch_types=(
            [pltpu.VMEM((CAPBUF,), jnp.int32),
             pltpu.VMEM((CAPBUF,), jnp.int32)]
            + bufset + bufset
            + [pltpu.VMEM((CHUNK, DH), jnp.float32),
               pltpu.VMEM((16,), jnp.int32),
               pltpu.VMEM_SHARED((HALF + ROWBLK, DH), jnp.float32),
               pltpu.SemaphoreType.DMA,
               pltpu.SemaphoreType.DMA]
        ),
    )
    return fn(k0, q0, v0, s0, k1, q1, v1, s1, srcp, dstp, cnt)


def kernel(x, edge_index, edge_attr, W_key, b_key, W_query, b_query,
           W_value, b_value, W_skip, b_skip, bias):
    del edge_attr
    k0, k1, q0, q1, v0, v1, s0, s1 = _projections(
        x, W_key.T, W_query.T, W_value.T, W_skip.T,
        b_key.reshape(1, D), b_query.reshape(1, D), b_value.reshape(1, D),
        b_skip.reshape(1, D), bias.reshape(1, D))
    src = jnp.pad(edge_index[0], (0, 16))
    dst = jnp.pad(edge_index[1], (0, 16))
    srcp, dstp, cnt = _partition(src, dst)
    out0, out1 = _edge_aggregate(k0, q0, v0, s0, k1, q1, v1, s1,
                                 srcp, dstp, cnt)
    return jnp.concatenate([out0, out1], axis=1)


# speedup vs baseline: 1.1930x; 1.1314x over previous
import functools

import jax
import jax.numpy as jnp
from jax import lax
from jax.experimental import pallas as pl
from jax.experimental.pallas import tpu as pltpu
from jax.experimental.pallas import tpu_sc as plsc

N = 10000
E = 160000
D = 256
DH = D // 2

NUM_CORES = 2
NUM_SUBCORES = 16
NSEG = NUM_CORES * NUM_SUBCORES
HALF = N // NUM_CORES
SEG = E // NSEG
SEGCAP = SEG + 8
CHUNK = 80
ROWBLK = 8
NROWCHUNK = (HALF + ROWBLK - 1) // ROWBLK

def _ones16():
    return jnp.ones((16,), jnp.int32)


def _zeros16():
    return jnp.zeros((16,), jnp.int32)


def _proj_body(x_ref, wk_ref, wq_ref, wv_ref, ws_ref, bk_ref, bq_ref,
               bv_ref, bs_ref, bias_ref,
               k0_ref, k1_ref, q0_ref, q1_ref, v0_ref, v1_ref,
               s0_ref, s1_ref):
    xb = x_ref[...]
    k = jnp.dot(xb, wk_ref[...], preferred_element_type=jnp.float32) + bk_ref[...]
    q = jnp.dot(xb, wq_ref[...], preferred_element_type=jnp.float32) + bq_ref[...]
    v = jnp.dot(xb, wv_ref[...], preferred_element_type=jnp.float32) + bv_ref[...]
    s = (jnp.dot(xb, ws_ref[...], preferred_element_type=jnp.float32)
         + bs_ref[...] + bias_ref[...])
    k0_ref[...] = k[:, :DH]
    k1_ref[...] = k[:, DH:]
    q0_ref[...] = q[:, :DH]
    q1_ref[...] = q[:, DH:]
    v0_ref[...] = v[:, :DH]
    v1_ref[...] = v[:, DH:]
    s0_ref[...] = s[:, :DH]
    s1_ref[...] = s[:, DH:]


def _projections(x, wkT, wqT, wvT, wsT, bk, bq, bv, bs, bias):
    blk = 1000
    grid = (N // blk,)
    xspec = pl.BlockSpec((blk, D), lambda i: (i, 0))
    wspec = pl.BlockSpec((D, D), lambda i: (0, 0))
    bspec = pl.BlockSpec((1, D), lambda i: (0, 0))
    ospec = pl.BlockSpec((blk, DH), lambda i: (i, 0))
    oshape = jax.ShapeDtypeStruct((N, DH), jnp.float32)
    return pl.pallas_call(
        _proj_body,
        grid=grid,
        in_specs=[xspec, wspec, wspec, wspec, wspec,
                  bspec, bspec, bspec, bspec, bspec],
        out_specs=[ospec] * 8,
        out_shape=[oshape] * 8,
    )(x, wkT, wqT, wvT, wsT, bk, bq, bv, bs, bias)


def _part_body(src_hbm, dst_hbm, srcp_hbm, dstp_hbm, cnt_hbm,
               src_seg, dst_seg, osrc0, odst0, osrc1, odst1, cbuf):
    c = lax.axis_index("c")
    s = lax.axis_index("s")
    seg = c * NUM_SUBCORES + s
    e0 = seg * SEG

    pltpu.sync_copy(src_hbm.at[pl.ds(e0, SEGCAP)], src_seg)
    pltpu.sync_copy(dst_hbm.at[pl.ds(e0, SEGCAP)], dst_seg)

    def route(d, sv, valid, f0v, f1v):
        m0 = d < HALF
        m1 = d >= HALF
        if valid is not None:
            m0 = valid & m0
            m1 = valid & m1
        i0 = jnp.where(m0, _ones16(), _zeros16())
        i1 = jnp.where(m1, _ones16(), _zeros16())
        p0 = f0v + lax.cumsum(i0) - 1
        p1 = f1v + lax.cumsum(i1) - 1
        plsc.store_scatter(odst0, [p0], d, mask=m0)
        plsc.store_scatter(osrc0, [p0], sv, mask=m0)
        plsc.store_scatter(odst1, [p1], d - HALF, mask=m1)
        plsc.store_scatter(osrc1, [p1], sv, mask=m1)
        return (f0v + plsc.all_reduce_population_count(m0),
                f1v + plsc.all_reduce_population_count(m1))

    def step(i, carry):
        f0v, f1v = carry
        sl = pl.ds(i * 16, 16)
        return route(dst_seg[sl], src_seg[sl], None, f0v, f1v)

    nfull = SEG // 16
    f0v, f1v = lax.fori_loop(0, nfull, step, (_zeros16(), _zeros16()))

    tail = SEG - nfull * 16
    if tail:
        sl = pl.ds(nfull * 16, 16)
        valid = lax.iota(jnp.int32, 16) < tail
        f0v, f1v = route(dst_seg[sl], src_seg[sl], valid, f0v, f1v)

    pltpu.sync_copy(osrc0, srcp_hbm.at[pl.ds(seg * SEGCAP, SEGCAP)])
    pltpu.sync_copy(odst0, dstp_hbm.at[pl.ds(seg * SEGCAP, SEGCAP)])
    pltpu.sync_copy(osrc1, srcp_hbm.at[pl.ds((NSEG + seg) * SEGCAP, SEGCAP)])
    pltpu.sync_copy(odst1, dstp_hbm.at[pl.ds((NSEG + seg) * SEGCAP, SEGCAP)])
    cbuf[pl.ds(0, 16)] = f0v
    pltpu.sync_copy(cbuf, cnt_hbm.at[pl.ds(seg * 16, 16)])
    cbuf[pl.ds(0, 16)] = f1v
    pltpu.sync_copy(cbuf, cnt_hbm.at[pl.ds((NSEG + seg) * 16, 16)])


def _partition(src, dst):
    mesh = plsc.VectorSubcoreMesh(core_axis_name="c", subcore_axis_name="s",
                                  num_cores=NUM_CORES,
                                  num_subcores=NUM_SUBCORES)
    fn = pl.kernel(
        _part_body,
        compiler_params=pltpu.CompilerParams(needs_layout_passes=False),
        out_type=[
            jax.ShapeDtypeStruct((2 * NSEG * SEGCAP,), jnp.int32),
            jax.ShapeDtypeStruct((2 * NSEG * SEGCAP,), jnp.int32),
            jax.ShapeDtypeStruct((2 * NSEG * 16,), jnp.int32),
        ],
        mesh=mesh,
        scratch_types=[
            pltpu.VMEM((SEGCAP,), jnp.int32),
            pltpu.VMEM((SEGCAP,), jnp.int32),
            pltpu.VMEM((SEGCAP,), jnp.int32),
            pltpu.VMEM((SEGCAP,), jnp.int32),
            pltpu.VMEM((SEGCAP,), jnp.int32),
            pltpu.VMEM((SEGCAP,), jnp.int32),
            pltpu.VMEM((16,), jnp.int32),
        ],
    )
    return fn(src, dst)


def _edge_body(k_hbm, q_hbm, v_hbm, skip_hbm, srcp_hbm, dstp_hbm, cnt_hbm,
               out_hbm, src_seg, dst_seg,
               idx_src0, idx_dstg0, idx_loc0, kbuf0, qbuf0, vbuf0,
               idx_src1, idx_dstg1, idx_loc1, kbuf1, qbuf1, vbuf1,
               mbuf, cbuf, acc, sem0, sem1):
    c = lax.axis_index("c")
    s = lax.axis_index("s")
    base = c * HALF

    def init_step(t, _):
        chunk = s + t * NUM_SUBCORES

        @pl.when(chunk < NROWCHUNK)
        def _():
            pltpu.sync_copy(skip_hbm.at[pl.ds(base + chunk * ROWBLK, ROWBLK)],
                            acc.at[pl.ds(chunk * ROWBLK, ROWBLK)])
        return 0

    lax.fori_loop(0, (NROWCHUNK + NUM_SUBCORES - 1) // NUM_SUBCORES,
                  init_step, 0)
    plsc.subcore_barrier()

    lanes = [lax.iota(jnp.int32, 16) + j * 16 for j in range(CHUNK // 16)]
    sets = ((idx_src0, idx_dstg0, idx_loc0, kbuf0, qbuf0, vbuf0, sem0),
            (idx_src1, idx_dstg1, idx_loc1, kbuf1, qbuf1, vbuf1, sem1))

    def build(blk, remv, st):
        idx_src, idx_dstg, idx_loc = st[0], st[1], st[2]
        for j in range(CHUNK // 16):
            sl = pl.ds(blk * CHUNK + j * 16, 16)
            osl = pl.ds(j * 16, 16)
            valid = lanes[j] < remv
            sv = src_seg[sl]
            dv = dst_seg[sl]
            idx_src[osl] = jnp.where(valid, sv, 0)
            idx_dstg[osl] = jnp.where(valid, dv + base, 0)
            idx_loc[osl] = jnp.where(valid, dv, HALF)

    def fire(st):
        idx_src, idx_dstg, st_sem = st[0], st[1], st[6]
        pltpu.make_async_copy(k_hbm.at[idx_dstg], st[3], st_sem).start()
        pltpu.make_async_copy(q_hbm.at[idx_src], st[4], st_sem).start()
        pltpu.make_async_copy(v_hbm.at[idx_src], st[5], st_sem).start()

    def wait3(st):
        idx_src, idx_dstg, st_sem = st[0], st[1], st[6]
        pltpu.make_async_copy(k_hbm.at[idx_dstg], st[3], st_sem).wait()
        pltpu.make_async_copy(q_hbm.at[idx_src], st[4], st_sem).wait()
        pltpu.make_async_copy(v_hbm.at[idx_src], st[5], st_sem).wait()

    def process(st):
        kbuf, qbuf, vbuf = st[3], st[4], st[5]

        def gate_step(e, _):
            for j in range(DH // 16):
                sl = pl.ds(j * 16, 16)
                t = kbuf[e, sl] + qbuf[e, sl]
                sig = 1.0 / (1.0 + jnp.exp(-t))
                mbuf[e, sl] = sig * vbuf[e, sl]
            return 0

        lax.fori_loop(0, CHUNK, gate_step, 0)
        pltpu.sync_copy(mbuf, acc.at[st[2]], add=True)

    for t2 in range(2):
        seg = 2 * s + t2
        lbase = (c * NSEG + seg) * SEGCAP
        pltpu.sync_copy(srcp_hbm.at[pl.ds(lbase, SEGCAP)],
                        src_seg.at[pl.ds(0, SEGCAP)])
        pltpu.sync_copy(dstp_hbm.at[pl.ds(lbase, SEGCAP)],
                        dst_seg.at[pl.ds(0, SEGCAP)])
        pltpu.sync_copy(cnt_hbm.at[pl.ds((c * NSEG + seg) * 16, 16)], cbuf)
        cntv = cbuf[pl.ds(0, 16)]
        cnt = jnp.max(cntv)
        nblk = (cnt + (CHUNK - 1)) // CHUNK
        npair = (nblk + 1) // 2

        @pl.when(nblk > 0)
        def _():
            build(0, cntv, sets[0])
            fire(sets[0])

        def pair_step(p, remv):
            for half in range(2):
                st = sets[half]
                other = sets[1 - half]
                blk = p * 2 + half
                rv = remv

                @pl.when(blk + 1 < nblk)
                def _():
                    build(blk + 1, rv, other)
                    fire(other)

                @pl.when(blk < nblk)
                def _():
                    wait3(st)
                    process(st)

                remv = remv - CHUNK
            return remv

        lax.fori_loop(0, npair, pair_step, cntv - CHUNK)

    plsc.subcore_barrier()

    def out_step(t, _):
        chunk = s + t * NUM_SUBCORES

        @pl.when(chunk < NROWCHUNK)
        def _():
            pltpu.sync_copy(acc.at[pl.ds(chunk * ROWBLK, ROWBLK)],
                            out_hbm.at[pl.ds(base + chunk * ROWBLK, ROWBLK)])
        return 0

    lax.fori_loop(0, (NROWCHUNK + NUM_SUBCORES - 1) // NUM_SUBCORES,
                  out_step, 0)


CAPBUF = ((SEG + CHUNK - 1) // CHUNK) * CHUNK


def _edge_aggregate(k, q, v, skip, srcp, dstp, cnt):
    mesh = plsc.VectorSubcoreMesh(core_axis_name="c", subcore_axis_name="s",
                                  num_cores=NUM_CORES,
                                  num_subcores=NUM_SUBCORES)
    bufset = [
        pltpu.VMEM((CHUNK,), jnp.int32),
        pltpu.VMEM((CHUNK,), jnp.int32),
        pltpu.VMEM((CHUNK,), jnp.int32),
        pltpu.VMEM((CHUNK, DH), jnp.float32),
        pltpu.VMEM((CHUNK, DH), jnp.float32),
        pltpu.VMEM((CHUNK, DH), jnp.float32),
    ]
    fn = pl.kernel(
        _edge_body,
        compiler_params=pltpu.CompilerParams(needs_layout_passes=False),
        out_type=jax.ShapeDtypeStruct((N, DH), jnp.float32),
        mesh=mesh,
        scratch_types=(
            [pltpu.VMEM((CAPBUF,), jnp.int32),
             pltpu.VMEM((CAPBUF,), jnp.int32)]
            + bufset + bufset
            + [pltpu.VMEM((CHUNK, DH), jnp.float32),
               pltpu.VMEM((16,), jnp.int32),
               pltpu.VMEM_SHARED((HALF + ROWBLK, DH), jnp.float32),
               pltpu.SemaphoreType.DMA,
               pltpu.SemaphoreType.DMA]
        ),
    )
    return fn(k, q, v, skip, srcp, dstp, cnt)


def kernel(x, edge_index, edge_attr, W_key, b_key, W_query, b_query,
           W_value, b_value, W_skip, b_skip, bias):
    del edge_attr
    k0, k1, q0, q1, v0, v1, s0, s1 = _projections(
        x, W_key.T, W_query.T, W_value.T, W_skip.T,
        b_key.reshape(1, D), b_query.reshape(1, D), b_value.reshape(1, D),
        b_skip.reshape(1, D), bias.reshape(1, D))
    src = jnp.pad(edge_index[0], (0, 16))
    dst = jnp.pad(edge_index[1], (0, 16))
    srcp, dstp, cnt = _partition(src, dst)
    out0 = _edge_aggregate(k0, q0, v0, s0, srcp, dstp, cnt)
    out1 = _edge_aggregate(k1, q1, v1, s1, srcp, dstp, cnt)
    return jnp.concatenate([out0, out1], axis=1)


# speedup vs baseline: 1.2019x; 1.0074x over previous
import functools

import jax
import jax.numpy as jnp
from jax import lax
from jax.experimental import pallas as pl
from jax.experimental.pallas import tpu as pltpu
from jax.experimental.pallas import tpu_sc as plsc

N = 10000
E = 160000
D = 256
DH = D // 2

NUM_CORES = 2
NUM_SUBCORES = 16
NSEG = NUM_CORES * NUM_SUBCORES
HALF = N // NUM_CORES
SEG = E // NSEG
SEGCAP = SEG + 8
CHUNK = 80
ROWBLK = 8
NROWCHUNK = (HALF + ROWBLK - 1) // ROWBLK

def _ones16():
    return jnp.ones((16,), jnp.int32)


def _zeros16():
    return jnp.zeros((16,), jnp.int32)


def _proj_body(x_ref, wk_ref, wq_ref, wv_ref, ws_ref, bk_ref, bq_ref,
               bv_ref, bs_ref, bias_ref,
               k0_ref, k1_ref, q0_ref, q1_ref, v0_ref, v1_ref,
               s0_ref, s1_ref):
    xb = x_ref[...]
    k = jnp.dot(xb, wk_ref[...], preferred_element_type=jnp.float32) + bk_ref[...]
    q = jnp.dot(xb, wq_ref[...], preferred_element_type=jnp.float32) + bq_ref[...]
    v = jnp.dot(xb, wv_ref[...], preferred_element_type=jnp.float32) + bv_ref[...]
    s = (jnp.dot(xb, ws_ref[...], preferred_element_type=jnp.float32)
         + bs_ref[...] + bias_ref[...])
    kb = k.astype(jnp.bfloat16)
    qb = q.astype(jnp.bfloat16)
    vb = v.astype(jnp.bfloat16)
    k0_ref[...] = kb[:, :DH]
    k1_ref[...] = kb[:, DH:]
    q0_ref[...] = qb[:, :DH]
    q1_ref[...] = qb[:, DH:]
    v0_ref[...] = vb[:, :DH]
    v1_ref[...] = vb[:, DH:]
    s0_ref[...] = s[:, :DH]
    s1_ref[...] = s[:, DH:]


def _projections(x, wkT, wqT, wvT, wsT, bk, bq, bv, bs, bias):
    blk = 1000
    grid = (N // blk,)
    xspec = pl.BlockSpec((blk, D), lambda i: (i, 0))
    wspec = pl.BlockSpec((D, D), lambda i: (0, 0))
    bspec = pl.BlockSpec((1, D), lambda i: (0, 0))
    ospec = pl.BlockSpec((blk, DH), lambda i: (i, 0))
    obf = jax.ShapeDtypeStruct((N, DH), jnp.bfloat16)
    of32 = jax.ShapeDtypeStruct((N, DH), jnp.float32)
    return pl.pallas_call(
        _proj_body,
        grid=grid,
        in_specs=[xspec, wspec, wspec, wspec, wspec,
                  bspec, bspec, bspec, bspec, bspec],
        out_specs=[ospec] * 8,
        out_shape=[obf] * 6 + [of32] * 2,
    )(x, wkT, wqT, wvT, wsT, bk, bq, bv, bs, bias)


def _part_body(src_hbm, dst_hbm, srcp_hbm, dstp_hbm, cnt_hbm,
               src_seg, dst_seg, osrc0, odst0, osrc1, odst1, cbuf):
    c = lax.axis_index("c")
    s = lax.axis_index("s")
    seg = c * NUM_SUBCORES + s
    e0 = seg * SEG

    pltpu.sync_copy(src_hbm.at[pl.ds(e0, SEGCAP)], src_seg)
    pltpu.sync_copy(dst_hbm.at[pl.ds(e0, SEGCAP)], dst_seg)

    def route(d, sv, valid, f0v, f1v):
        m0 = d < HALF
        m1 = d >= HALF
        if valid is not None:
            m0 = valid & m0
            m1 = valid & m1
        i0 = jnp.where(m0, _ones16(), _zeros16())
        i1 = jnp.where(m1, _ones16(), _zeros16())
        p0 = f0v + lax.cumsum(i0) - 1
        p1 = f1v + lax.cumsum(i1) - 1
        plsc.store_scatter(odst0, [p0], d, mask=m0)
        plsc.store_scatter(osrc0, [p0], sv, mask=m0)
        plsc.store_scatter(odst1, [p1], d - HALF, mask=m1)
        plsc.store_scatter(osrc1, [p1], sv, mask=m1)
        return (f0v + plsc.all_reduce_population_count(m0),
                f1v + plsc.all_reduce_population_count(m1))

    def step(i, carry):
        f0v, f1v = carry
        sl = pl.ds(i * 16, 16)
        return route(dst_seg[sl], src_seg[sl], None, f0v, f1v)

    nfull = SEG // 16
    f0v, f1v = lax.fori_loop(0, nfull, step, (_zeros16(), _zeros16()))

    tail = SEG - nfull * 16
    if tail:
        sl = pl.ds(nfull * 16, 16)
        valid = lax.iota(jnp.int32, 16) < tail
        f0v, f1v = route(dst_seg[sl], src_seg[sl], valid, f0v, f1v)

    pltpu.sync_copy(osrc0, srcp_hbm.at[pl.ds(seg * SEGCAP, SEGCAP)])
    pltpu.sync_copy(odst0, dstp_hbm.at[pl.ds(seg * SEGCAP, SEGCAP)])
    pltpu.sync_copy(osrc1, srcp_hbm.at[pl.ds((NSEG + seg) * SEGCAP, SEGCAP)])
    pltpu.sync_copy(odst1, dstp_hbm.at[pl.ds((NSEG + seg) * SEGCAP, SEGCAP)])
    cbuf[pl.ds(0, 16)] = f0v
    pltpu.sync_copy(cbuf, cnt_hbm.at[pl.ds(seg * 16, 16)])
    cbuf[pl.ds(0, 16)] = f1v
    pltpu.sync_copy(cbuf, cnt_hbm.at[pl.ds((NSEG + seg) * 16, 16)])


def _partition(src, dst):
    mesh = plsc.VectorSubcoreMesh(core_axis_name="c", subcore_axis_name="s",
                                  num_cores=NUM_CORES,
                                  num_subcores=NUM_SUBCORES)
    fn = pl.kernel(
        _part_body,
        compiler_params=pltpu.CompilerParams(needs_layout_passes=False),
        out_type=[
            jax.ShapeDtypeStruct((2 * NSEG * SEGCAP,), jnp.int32),
            jax.ShapeDtypeStruct((2 * NSEG * SEGCAP,), jnp.int32),
            jax.ShapeDtypeStruct((2 * NSEG * 16,), jnp.int32),
        ],
        mesh=mesh,
        scratch_types=[
            pltpu.VMEM((SEGCAP,), jnp.int32),
            pltpu.VMEM((SEGCAP,), jnp.int32),
            pltpu.VMEM((SEGCAP,), jnp.int32),
            pltpu.VMEM((SEGCAP,), jnp.int32),
            pltpu.VMEM((SEGCAP,), jnp.int32),
            pltpu.VMEM((SEGCAP,), jnp.int32),
            pltpu.VMEM((16,), jnp.int32),
        ],
    )
    return fn(src, dst)


def _edge_body(koff, kk_hbm, qv_hbm, skip_hbm, srcp_hbm, dstp_hbm, cnt_hbm,
               out_hbm, src_seg, dst_seg,
               idx_src0, idx_dstg0, idx_loc0, kbuf0, qbuf0,
               idx_src1, idx_dstg1, idx_loc1, kbuf1, qbuf1,
               mbuf, cbuf, acc, sem0, sem1):
    c = lax.axis_index("c")
    s = lax.axis_index("s")
    base = c * HALF

    def init_step(t, _):
        chunk = s + t * NUM_SUBCORES

        @pl.when(chunk < NROWCHUNK)
        def _():
            pltpu.sync_copy(skip_hbm.at[pl.ds(base + chunk * ROWBLK, ROWBLK)],
                            acc.at[pl.ds(chunk * ROWBLK, ROWBLK)])
        return 0

    lax.fori_loop(0, (NROWCHUNK + NUM_SUBCORES - 1) // NUM_SUBCORES,
                  init_step, 0)
    plsc.subcore_barrier()

    lanes = [lax.iota(jnp.int32, 16) + j * 16 for j in range(CHUNK // 16)]
    sets = ((idx_src0, idx_dstg0, idx_loc0, kbuf0, qbuf0, sem0),
            (idx_src1, idx_dstg1, idx_loc1, kbuf1, qbuf1, sem1))

    def build(blk, remv, st):
        idx_src, idx_dstg, idx_loc = st[0], st[1], st[2]
        for j in range(CHUNK // 16):
            sl = pl.ds(blk * CHUNK + j * 16, 16)
            osl = pl.ds(j * 16, 16)
            valid = lanes[j] < remv
            sv = src_seg[sl]
            dv = dst_seg[sl]
            idx_src[osl] = jnp.where(valid, sv, 0)
            idx_dstg[osl] = jnp.where(valid, dv + base, 0)
            idx_loc[osl] = jnp.where(valid, dv, HALF)

    def fire(st):
        idx_src, idx_dstg, st_sem = st[0], st[1], st[5]
        pltpu.make_async_copy(kk_hbm.at[idx_dstg], st[3], st_sem).start()
        pltpu.make_async_copy(qv_hbm.at[idx_src], st[4], st_sem).start()

    def wait3(st):
        idx_src, idx_dstg, st_sem = st[0], st[1], st[5]
        pltpu.make_async_copy(kk_hbm.at[idx_dstg], st[3], st_sem).wait()
        pltpu.make_async_copy(qv_hbm.at[idx_src], st[4], st_sem).wait()

    def process(st):
        kkbuf, qvbuf = st[3], st[4]

        def gate_step(e, _):
            for j in range(DH // 32):
                kw = plsc.bitcast(kkbuf[e, pl.ds(koff + j * 16, 16)],
                                  jnp.bfloat16)
                qw = plsc.bitcast(qvbuf[e, pl.ds(j * 16, 16)], jnp.bfloat16)
                vw = plsc.bitcast(qvbuf[e, pl.ds(DH // 2 + j * 16, 16)],
                                  jnp.bfloat16)
                tb = kw + qw
                ta, tb2 = plsc.unpack(tb, format=plsc.PackFormat.INTERLEAVED)
                va, vb2 = plsc.unpack(vw, format=plsc.PackFormat.INTERLEAVED)
                siga = 1.0 / (1.0 + jnp.exp(-ta))
                sigb = 1.0 / (1.0 + jnp.exp(-tb2))
                mbuf[e, pl.ds(j * 32, 16)] = siga * va
                mbuf[e, pl.ds(j * 32 + 16, 16)] = sigb * vb2
            return 0

        lax.fori_loop(0, CHUNK, gate_step, 0)
        pltpu.sync_copy(mbuf, acc.at[st[2]], add=True)

    for t2 in range(2):
        seg = 2 * s + t2
        lbase = (c * NSEG + seg) * SEGCAP
        pltpu.sync_copy(srcp_hbm.at[pl.ds(lbase, SEGCAP)],
                        src_seg.at[pl.ds(0, SEGCAP)])
        pltpu.sync_copy(dstp_hbm.at[pl.ds(lbase, SEGCAP)],
                        dst_seg.at[pl.ds(0, SEGCAP)])
        pltpu.sync_copy(cnt_hbm.at[pl.ds((c * NSEG + seg) * 16, 16)], cbuf)
        cntv = cbuf[pl.ds(0, 16)]
        cnt = jnp.max(cntv)
        nblk = (cnt + (CHUNK - 1)) // CHUNK
        npair = (nblk + 1) // 2

        @pl.when(nblk > 0)
        def _():
            build(0, cntv, sets[0])
            fire(sets[0])

        def pair_step(p, remv):
            for half in range(2):
                st = sets[half]
                other = sets[1 - half]
                blk = p * 2 + half
                rv = remv

                @pl.when(blk + 1 < nblk)
                def _():
                    build(blk + 1, rv, other)
                    fire(other)

                @pl.when(blk < nblk)
                def _():
                    wait3(st)
                    process(st)

                remv = remv - CHUNK
            return remv

        lax.fori_loop(0, npair, pair_step, cntv - CHUNK)

    plsc.subcore_barrier()

    def out_step(t, _):
        chunk = s + t * NUM_SUBCORES

        @pl.when(chunk < NROWCHUNK)
        def _():
            pltpu.sync_copy(acc.at[pl.ds(chunk * ROWBLK, ROWBLK)],
                            out_hbm.at[pl.ds(base + chunk * ROWBLK, ROWBLK)])
        return 0

    lax.fori_loop(0, (NROWCHUNK + NUM_SUBCORES - 1) // NUM_SUBCORES,
                  out_step, 0)


CAPBUF = ((SEG + CHUNK - 1) // CHUNK) * CHUNK


def _edge_aggregate(koff, kk, qv, skip, srcp, dstp, cnt):
    mesh = plsc.VectorSubcoreMesh(core_axis_name="c", subcore_axis_name="s",
                                  num_cores=NUM_CORES,
                                  num_subcores=NUM_SUBCORES)
    bufset = [
        pltpu.VMEM((CHUNK,), jnp.int32),
        pltpu.VMEM((CHUNK,), jnp.int32),
        pltpu.VMEM((CHUNK,), jnp.int32),
        pltpu.VMEM((CHUNK, D // 2), jnp.int32),
        pltpu.VMEM((CHUNK, D // 2), jnp.int32),
    ]
    fn = pl.kernel(
        functools.partial(_edge_body, koff),
        compiler_params=pltpu.CompilerParams(needs_layout_passes=False),
        out_type=jax.ShapeDtypeStruct((N, DH), jnp.float32),
        mesh=mesh,
        scratch_types=(
            [pltpu.VMEM((CAPBUF,), jnp.int32),
             pltpu.VMEM((CAPBUF,), jnp.int32)]
            + bufset + bufset
            + [pltpu.VMEM((CHUNK, DH), jnp.float32),
               pltpu.VMEM((16,), jnp.int32),
               pltpu.VMEM_SHARED((HALF + ROWBLK, DH), jnp.float32),
               pltpu.SemaphoreType.DMA,
               pltpu.SemaphoreType.DMA]
        ),
    )
    return fn(kk, qv, skip, srcp, dstp, cnt)


def kernel(x, edge_index, edge_attr, W_key, b_key, W_query, b_query,
           W_value, b_value, W_skip, b_skip, bias):
    del edge_attr
    k0, k1, q0, q1, v0, v1, s0, s1 = _projections(
        x, W_key.T, W_query.T, W_value.T, W_skip.T,
        b_key.reshape(1, D), b_query.reshape(1, D), b_value.reshape(1, D),
        b_skip.reshape(1, D), bias.reshape(1, D))

    def _shuf(a):
        return a.reshape(N, DH // 32, 2, 16).swapaxes(2, 3).reshape(N, DH)

    def _pack(a, b):
        ab = jnp.concatenate([_shuf(a), _shuf(b)], axis=1)
        return jax.lax.bitcast_convert_type(ab.reshape(N, D // 2, 2),
                                            jnp.int32)

    kk = _pack(k0, k1)
    qv0 = _pack(q0, v0)
    qv1 = _pack(q1, v1)
    src = jnp.pad(edge_index[0], (0, 16))
    dst = jnp.pad(edge_index[1], (0, 16))
    srcp, dstp, cnt = _partition(src, dst)
    out0 = _edge_aggregate(0, kk, qv0, s0, srcp, dstp, cnt)
    out1 = _edge_aggregate(DH // 2, kk, qv1, s1, srcp, dstp, cnt)
    return jnp.concatenate([out0, out1], axis=1)
